# baseline (speedup 1.0000x reference)
import functools

import jax
import jax.numpy as jnp
from jax import lax
from jax.experimental import pallas as pl
from jax.experimental.pallas import tpu as pltpu
from jax.experimental.pallas import tpu_sc as plsc

mesh = plsc.VectorSubcoreMesh(core_axis_name="c", subcore_axis_name="s")

NBLK = 246


@functools.partial(
    pl.kernel,
    mesh=mesh,
    out_type=jax.ShapeDtypeStruct((200, 8, 32, 8, 128), jnp.float32),
    scratch_types=[
        pltpu.HBM((1000064, 64), jnp.float32),
        pltpu.VMEM((2, 8, 8, 128), jnp.float32),
        pltpu.VMEM((2, 128, 64), jnp.float32),
        pltpu.SemaphoreType.DMA((2,)),
        pltpu.SemaphoreType.DMA((2,)),
        pltpu.VMEM((16,), jnp.float32),
    ],
    compiler_params=pltpu.CompilerParams(
        use_tc_tiling_on_sc=False,
        disable_bounds_checks=True,
        has_side_effects=True,
        needs_layout_passes=False,
    ),
)
def _conv(tview_hbm, out_hbm, trm_hbm, tbuf, rbuf, gsem, wsem, buf_v):
    wid = lax.axis_index("s") * 2 + lax.axis_index("c")
    b0 = wid * NBLK
    lane = lax.iota(jnp.int32, 16)
    rowg = [lane + (g * 16) for g in range(8)]
    zeros16 = jnp.zeros((16,), jnp.int32)

    def blk(i):
        return jnp.minimum(b0 + i, 7812)

    def stage_all(i, b):
        vb = blk(i)
        for d8 in range(8):
            pltpu.make_async_copy(
                tview_hbm.at[d8, vb], tbuf.at[b, d8], gsem.at[b]
            ).start()

    def stage_wait(i, b):
        vb = blk(i)
        for d8 in range(8):
            pltpu.make_async_copy(
                tview_hbm.at[d8, vb], tbuf.at[b, d8], gsem.at[b]
            ).wait()

    def write_desc(i, b):
        return pltpu.make_async_copy(
            rbuf.at[b], trm_hbm.at[pl.ds(blk(i) * 128, 128)], wsem.at[b]
        )

    def transpose(b):
        for d8 in range(8):
            for ds in range(8):
                col = zeros16 + (d8 * 8 + ds)
                for g in range(8):
                    v = tbuf[b, d8, ds, pl.ds(g * 16, 16)]
                    plsc.store_scatter(rbuf.at[b], [rowg[g], col], v)

    for b in range(2):
        stage_all(b, b)

    def body(j, carry):
        for b in range(2):
            i = 2 * j + b
            stage_wait(i, b)

            @pl.when(i >= 2)
            def _():
                write_desc(i - 2, b).wait()

            transpose(b)
            write_desc(i, b).start()

            @pl.when(i + 2 < NBLK)
            def _():
                stage_all(i + 2, b)

        return carry

    lax.fori_loop(0, NBLK // 2, body, 0)

    for b in range(2):
        write_desc(NBLK - 2 + b, b).wait()

    @pl.when(wid == 0)
    def _():
        pltpu.sync_copy(trm_hbm.at[0, pl.ds(0, 16)], buf_v)
        pltpu.sync_copy(buf_v, out_hbm.at[0, 0, 0, 0, pl.ds(0, 16)])


@jax.jit
def kernel(token_ids, embedding_weight):
    table = jnp.pad(embedding_weight, ((0, 64), (0, 0)))
    tview = table.T.reshape(8, 8, 7813, 128).transpose(0, 2, 1, 3)
    out5 = _conv(tview)
    return (
        out5.transpose(0, 1, 3, 2, 4)
        .reshape(200, 64, 4096)
        .transpose(2, 0, 1)
    )


# speedup vs baseline: 1.1467x; 1.1467x over previous
import functools

import jax
import jax.numpy as jnp
from jax import lax
from jax.experimental import pallas as pl
from jax.experimental.pallas import tpu as pltpu
from jax.experimental.pallas import tpu_sc as plsc

D_MODEL = 64
NUM_CORES = 2
NUM_SUBCORES = 16
NUM_WORKERS = NUM_CORES * NUM_SUBCORES

B_TOK = 4096
S_TOK = 200
SB = S_TOK // 8
BB = B_TOK // 128
KP = 4

TASKS = S_TOK * (BB // KP)
TASKS_PER_W = TASKS // NUM_WORKERS

mesh = plsc.VectorSubcoreMesh(core_axis_name="c", subcore_axis_name="s")


@functools.partial(
    pl.kernel,
    mesh=mesh,
    out_type=jax.ShapeDtypeStruct((B_TOK, S_TOK, D_MODEL), jnp.float32),
    scratch_types=[
        pltpu.VMEM((2, KP, 128), jnp.int32),
        pltpu.VMEM((2, KP * 128, D_MODEL), jnp.float32),
        pltpu.SemaphoreType.DMA((2,)),
        pltpu.SemaphoreType.DMA((2,)),
    ],
    compiler_params=pltpu.CompilerParams(
        use_tc_tiling_on_sc=False,
        skip_device_barrier=True,
        disable_bounds_checks=True,
        disable_semaphore_checks=True,
    ),
)
def _lookup(ids_hbm, table_hbm, out_hbm, idx_v, rows_v, gsem, wsem):
    wid = lax.axis_index("s") * NUM_CORES + lax.axis_index("c")
    t0 = wid * TASKS_PER_W

    def coords(t):
        s = t // (BB // KP)
        p = t % (BB // KP)
        return s // 8, s % 8, p

    def stage_ids(t, b):
        s8, ss, p = coords(t)
        pltpu.sync_copy(ids_hbm.at[s8, pl.ds(KP * p, KP), ss, :], idx_v.at[b])

    def gather_desc(b, k):
        return pltpu.make_async_copy(
            table_hbm.at[idx_v.at[b, k]],
            rows_v.at[b, pl.ds(k * 128, 128)],
            gsem.at[b],
        )

    def write_desc(t, b, k):
        s8, ss, p = coords(t)
        return pltpu.make_async_copy(
            rows_v.at[b, pl.ds(k * 128, 128)],
            out_hbm.at[pl.ds((KP * p + k) * 128, 128), s8 * 8 + ss, :],
            wsem.at[b],
        )

    for b in range(2):
        stage_ids(t0 + b, b)
        for k in range(KP):
            gather_desc(b, k).start()

    def body(j, carry):
        for b in range(2):
            t = t0 + 2 * j + b
            for k in range(KP):
                gather_desc(b, k).wait()
            for k in range(KP):
                write_desc(t, b, k).start()
        for b in range(2):
            t = t0 + 2 * j + b
            for k in range(KP):
                write_desc(t, b, k).wait()

            @pl.when(2 * j + b + 2 < TASKS_PER_W)
            def _():
                stage_ids(t + 2, b)
                for k in range(KP):
                    gather_desc(b, k).start()

        return carry

    lax.fori_loop(0, TASKS_PER_W // 2, body, 0)


@jax.jit
def kernel(token_ids, embedding_weight):
    ids_view = (
        token_ids.astype(jnp.int32)
        .T.reshape(SB, 8, BB, 128)
        .transpose(0, 2, 1, 3)
    )
    out = _lookup(ids_view, embedding_weight)
    return out
